# baseline (speedup 1.0000x reference)
import functools

import jax
import jax.numpy as jnp
import numpy as np
from jax import lax
from jax.experimental import pallas as pl
from jax.experimental.pallas import tpu as pltpu
from jax.experimental.pallas import tpu_sc as plsc

B, C, H, W = 4, 19, 512, 512
R = B * H
N = B * H * W
K = 100000
THRESH = np.float32(0.7)
THRESH_BITS = int(np.float32(0.7).view(np.int32))
ONE_BITS = int(np.float32(1.0).view(np.int32))


BH = 32
NHB = H // BH


def _dense_body(x1, x2, x3, x4, x5, t_ref, p_ref, f_ref):
    t = t_ref[...]
    cls = lax.broadcasted_iota(jnp.int32, (C, BH, W), 0)
    onehot = cls == t[None]
    ps = []
    fs = []
    for x in (x1, x2, x3, x4, x5):
        z = x[0]
        m = jnp.max(z, axis=0)
        s = jnp.sum(jnp.exp(z - m[None]), axis=0)
        lse = m + jnp.log(s)
        zt = jnp.sum(jnp.where(onehot, z, 0.0), axis=0)
        p = jnp.exp(zt - lse)
        ps.append(p)
        fs.append((1.0 - p) * (1.0 - p) * (lse - zt))
    p_ref[...] = jnp.stack(ps)
    f_ref[...] = jnp.stack(fs)


def _dense_stage(xs, t):
    x_spec = pl.BlockSpec((1, C, BH, W), lambda b, h: (b, 0, h, 0))
    t_spec = pl.BlockSpec((BH, W), lambda b, h: (b * NHB + h, 0))
    out_spec = pl.BlockSpec((5, BH, W), lambda b, h: (0, b * NHB + h, 0))
    out_shape = jax.ShapeDtypeStruct((5, R, W), jnp.float32)
    return pl.pallas_call(
        _dense_body,
        grid=(B, NHB),
        in_specs=[x_spec] * 5 + [t_spec],
        out_specs=[out_spec, out_spec],
        out_shape=[out_shape, out_shape],
    )(*xs, t)


NC, NS, L = 2, 16, 16
NW = NC * NS
WROWS = R // NW
CR = 16
NCH = WROWS // CR
NCHUNK = 5 * NCH


def _mine_sc_body(p_hbm, f_hbm, out_hbm, pb0, fb0, pb1, fb1, acc_v, sem0, sem1):
    wid = lax.axis_index("s") * NC + lax.axis_index("c")
    r0 = wid * WROWS
    bufs = ((pb0, fb0, sem0), (pb1, fb1, sem1))

    def chunk_src(idx):
        j = idx // NCH
        rr = r0 + (idx % NCH) * CR
        return p_hbm.at[j, pl.ds(rr, CR), :], f_hbm.at[j, pl.ds(rr, CR), :]

    def start(idx, par):
        psrc, fsrc = chunk_src(idx)
        pltpu.make_async_copy(psrc, bufs[par][0], bufs[par][2]).start()
        pltpu.make_async_copy(fsrc, bufs[par][1], bufs[par][2]).start()

    def wait(par):
        pltpu.make_async_copy(p_hbm.at[0, pl.ds(0, CR), :], bufs[par][0], bufs[par][2]).wait()
        pltpu.make_async_copy(f_hbm.at[0, pl.ds(0, CR), :], bufs[par][1], bufs[par][2]).wait()

    def compute(par, carry):
        pb, fb, _ = bufs[par]

        def vec_body(q, carry2):
            s7, c7 = carry2
            r = q >> 2
            cb = (q & 3) * 128
            for u in range(8):
                sl = pl.ds(cb + u * L, L)
                p = pb[r, sl]
                f = fb[r, sl]
                keep = p < THRESH
                s7 = s7 + jnp.where(keep, f, 0.0)
                c7 = c7 + jnp.where(keep, 1.0, 0.0)
            return s7, c7

        return lax.fori_loop(0, CR * 4, vec_body, carry)

    zero = jnp.zeros((L,), jnp.float32)
    start(0, 0)
    carry = (zero, zero)
    carrys = []
    for idx in range(NCHUNK):
        par = idx & 1
        start((idx + 1) % NCHUNK, 1 - par)
        wait(par)
        carry = compute(par, carry)
        if idx % NCH == NCH - 1:
            carrys.append(carry)
            carry = (zero, zero)
    wait(1 - (NCHUNK - 1) % 2)

    for j in range(5):
        acc_v[pl.ds(j * 2 * L, L)] = carrys[j][0]
        acc_v[pl.ds(j * 2 * L + L, L)] = carrys[j][1]
    pltpu.sync_copy(acc_v, out_hbm.at[pl.ds(wid * (5 * 2 * L), 5 * 2 * L)])


@functools.cache
def _make_mine_sc():
    mesh = plsc.VectorSubcoreMesh(
        core_axis_name="c", subcore_axis_name="s", num_cores=NC, num_subcores=NS
    )
    return functools.partial(
        pl.kernel,
        out_type=jax.ShapeDtypeStruct((NW * 5 * 2 * L,), jnp.float32),
        mesh=mesh,
        scratch_types=[
            pltpu.VMEM((CR, W), jnp.float32),
            pltpu.VMEM((CR, W), jnp.float32),
            pltpu.VMEM((CR, W), jnp.float32),
            pltpu.VMEM((CR, W), jnp.float32),
            pltpu.VMEM((5 * 2 * L,), jnp.float32),
            pltpu.SemaphoreType.DMA,
            pltpu.SemaphoreType.DMA,
        ],
    )(_mine_sc_body)


def _finish_body(pr_ref, out_ref):
    acc = pr_ref[...]
    loss = jnp.float32(0.0)
    need = jnp.float32(0.0)
    for j in range(5):
        s7 = jnp.sum(acc[:, j * 2 * L : j * 2 * L + L])
        c7 = jnp.sum(acc[:, j * 2 * L + L : (j + 1) * 2 * L])
        out_ref[2 + j] = s7
        out_ref[7 + j] = c7
        loss = loss + s7 / c7
        need = jnp.maximum(need, jnp.where(c7 <= K, 1.0, 0.0))
    out_ref[0] = loss
    out_ref[1] = need


def _finish(pr):
    return pl.pallas_call(
        _finish_body,
        in_specs=[pl.BlockSpec((NW, 5 * 2 * L), lambda: (0, 0))],
        out_specs=pl.BlockSpec(memory_space=pltpu.SMEM),
        out_shape=jax.ShapeDtypeStruct((16,), jnp.float32),
    )(pr.reshape(NW, 5 * 2 * L))


def _fb_body(p_ref, f_ref, out_ref, pb_ref):
    pb_ref[...] = lax.bitcast_convert_type(p_ref[0], jnp.int32)

    def bs_body(_, state):
        lo, hi = state
        mid = (lo + hi) // 2
        cnt = jnp.sum((pb_ref[...] <= mid).astype(jnp.int32))
        good = cnt >= K + 1
        return jnp.where(good, lo, mid + 1), jnp.where(good, mid, hi)

    _, vk_bits = lax.fori_loop(0, 31, bs_body, (jnp.int32(0), jnp.int32(ONE_BITS)))
    t_bits = jnp.maximum(vk_bits, jnp.int32(THRESH_BITS))
    keep = pb_ref[...] < t_bits
    s = jnp.sum(jnp.where(keep, f_ref[0], 0.0))
    c = jnp.sum(keep.astype(jnp.float32))
    lane = lax.broadcasted_iota(jnp.int32, (1, 128), 1)
    out_ref[0] = jnp.where(lane == 0, s, jnp.where(lane == 1, c, 0.0))


def _fallback(p, f):
    in_spec = pl.BlockSpec((1, R, W), lambda j: (j, 0, 0))
    return pl.pallas_call(
        _fb_body,
        grid=(5,),
        in_specs=[in_spec, in_spec],
        out_specs=pl.BlockSpec((1, 1, 128), lambda j: (j, 0, 0)),
        out_shape=jax.ShapeDtypeStruct((5, 1, 128), jnp.float32),
        scratch_shapes=[pltpu.VMEM((R, W), jnp.int32)],
    )(p, f)


def kernel(x1, x2, x3, x4, x5, target):
    t = target.astype(jnp.int32).reshape(R, W)
    p, f = _dense_stage((x1, x2, x3, x4, x5), t)
    pr = _make_mine_sc()(p, f)
    res = _finish(pr)

    def _common():
        return res[0]

    def _rare():
        fb = _fallback(p, f)[:, 0, :2]
        s7 = res[2:7]
        c7 = res[7:12]
        loss = jnp.where(c7 > K, s7 / c7, fb[:, 0] / fb[:, 1])
        return jnp.sum(loss)

    return lax.cond(res[1] > 0.0, _rare, _common)


# speedup vs baseline: 1.1024x; 1.1024x over previous
import functools

import jax
import jax.numpy as jnp
import numpy as np
from jax import lax
from jax.experimental import pallas as pl
from jax.experimental.pallas import tpu as pltpu
from jax.experimental.pallas import tpu_sc as plsc

B, C, H, W = 4, 19, 512, 512
R = B * H
N = B * H * W
K = 100000
THRESH = np.float32(0.7)
THRESH_BITS = int(np.float32(0.7).view(np.int32))
ONE_BITS = int(np.float32(1.0).view(np.int32))


BH = 128
NHB = H // BH


def _dense_body(x1, x2, x3, x4, x5, t_ref, p_ref, f_ref):
    t = t_ref[...]
    cls = lax.broadcasted_iota(jnp.int32, (C, BH, W), 0)
    onehot = cls == t[None]
    ps = []
    fs = []
    for x in (x1, x2, x3, x4, x5):
        z = x[0]
        m = jnp.max(z, axis=0)
        s = jnp.sum(jnp.exp(z - m[None]), axis=0)
        lse = m + jnp.log(s)
        zt = jnp.sum(jnp.where(onehot, z, 0.0), axis=0)
        p = jnp.exp(zt - lse)
        ps.append(p)
        fs.append((1.0 - p) * (1.0 - p) * (lse - zt))
    p_ref[...] = jnp.stack(ps)
    f_ref[...] = jnp.stack(fs)


def _dense_stage(xs, t):
    x_spec = pl.BlockSpec((1, C, BH, W), lambda b, h: (b, 0, h, 0))
    t_spec = pl.BlockSpec((BH, W), lambda b, h: (b * NHB + h, 0))
    out_spec = pl.BlockSpec((5, BH, W), lambda b, h: (0, b * NHB + h, 0))
    out_shape = jax.ShapeDtypeStruct((5, R, W), jnp.float32)
    return pl.pallas_call(
        _dense_body,
        grid=(B, NHB),
        in_specs=[x_spec] * 5 + [t_spec],
        out_specs=[out_spec, out_spec],
        out_shape=[out_shape, out_shape],
    )(*xs, t)


NC, NS, L = 2, 16, 16
NW = NC * NS
WROWS = R // NW
CR = 16
NCH = WROWS // CR
NCHUNK = 5 * NCH


def _mine_sc_body(p_hbm, f_hbm, out_hbm, pb0, fb0, pb1, fb1, acc_v, sem0, sem1):
    wid = lax.axis_index("s") * NC + lax.axis_index("c")
    r0 = wid * WROWS
    bufs = ((pb0, fb0, sem0), (pb1, fb1, sem1))

    def chunk_src(idx):
        j = idx // NCH
        rr = r0 + (idx % NCH) * CR
        return p_hbm.at[j, pl.ds(rr, CR), :], f_hbm.at[j, pl.ds(rr, CR), :]

    def start(idx, par):
        psrc, fsrc = chunk_src(idx)
        pltpu.make_async_copy(psrc, bufs[par][0], bufs[par][2]).start()
        pltpu.make_async_copy(fsrc, bufs[par][1], bufs[par][2]).start()

    def wait(par):
        pltpu.make_async_copy(p_hbm.at[0, pl.ds(0, CR), :], bufs[par][0], bufs[par][2]).wait()
        pltpu.make_async_copy(f_hbm.at[0, pl.ds(0, CR), :], bufs[par][1], bufs[par][2]).wait()

    def compute(par, carry):
        pb, fb, _ = bufs[par]

        def vec_body(q, carry2):
            s7, c7 = carry2
            r = q >> 2
            cb = (q & 3) * 128
            for u in range(8):
                sl = pl.ds(cb + u * L, L)
                p = pb[r, sl]
                f = fb[r, sl]
                keep = p < THRESH
                s7 = s7 + jnp.where(keep, f, 0.0)
                c7 = c7 + jnp.where(keep, 1.0, 0.0)
            return s7, c7

        return lax.fori_loop(0, CR * 4, vec_body, carry)

    zero = jnp.zeros((L,), jnp.float32)
    start(0, 0)
    carry = (zero, zero)
    carrys = []
    for idx in range(NCHUNK):
        par = idx & 1
        start((idx + 1) % NCHUNK, 1 - par)
        wait(par)
        carry = compute(par, carry)
        if idx % NCH == NCH - 1:
            carrys.append(carry)
            carry = (zero, zero)
    wait(1 - (NCHUNK - 1) % 2)

    for j in range(5):
        acc_v[pl.ds(j * 2 * L, L)] = carrys[j][0]
        acc_v[pl.ds(j * 2 * L + L, L)] = carrys[j][1]
    pltpu.sync_copy(acc_v, out_hbm.at[pl.ds(wid * (5 * 2 * L), 5 * 2 * L)])


@functools.cache
def _make_mine_sc():
    mesh = plsc.VectorSubcoreMesh(
        core_axis_name="c", subcore_axis_name="s", num_cores=NC, num_subcores=NS
    )
    return functools.partial(
        pl.kernel,
        out_type=jax.ShapeDtypeStruct((NW * 5 * 2 * L,), jnp.float32),
        mesh=mesh,
        scratch_types=[
            pltpu.VMEM((CR, W), jnp.float32),
            pltpu.VMEM((CR, W), jnp.float32),
            pltpu.VMEM((CR, W), jnp.float32),
            pltpu.VMEM((CR, W), jnp.float32),
            pltpu.VMEM((5 * 2 * L,), jnp.float32),
            pltpu.SemaphoreType.DMA,
            pltpu.SemaphoreType.DMA,
        ],
    )(_mine_sc_body)


def _finish_body(pr_ref, out_ref):
    acc = pr_ref[...]
    loss = jnp.float32(0.0)
    need = jnp.float32(0.0)
    for j in range(5):
        s7 = jnp.sum(acc[:, j * 2 * L : j * 2 * L + L])
        c7 = jnp.sum(acc[:, j * 2 * L + L : (j + 1) * 2 * L])
        out_ref[2 + j] = s7
        out_ref[7 + j] = c7
        loss = loss + s7 / c7
        need = jnp.maximum(need, jnp.where(c7 <= K, 1.0, 0.0))
    out_ref[0] = loss
    out_ref[1] = need


def _finish(pr):
    return pl.pallas_call(
        _finish_body,
        in_specs=[pl.BlockSpec((NW, 5 * 2 * L), lambda: (0, 0))],
        out_specs=pl.BlockSpec(memory_space=pltpu.SMEM),
        out_shape=jax.ShapeDtypeStruct((16,), jnp.float32),
    )(pr.reshape(NW, 5 * 2 * L))


def _fb_body(p_ref, f_ref, out_ref, pb_ref):
    pb_ref[...] = lax.bitcast_convert_type(p_ref[0], jnp.int32)

    def bs_body(_, state):
        lo, hi = state
        mid = (lo + hi) // 2
        cnt = jnp.sum((pb_ref[...] <= mid).astype(jnp.int32))
        good = cnt >= K + 1
        return jnp.where(good, lo, mid + 1), jnp.where(good, mid, hi)

    _, vk_bits = lax.fori_loop(0, 31, bs_body, (jnp.int32(0), jnp.int32(ONE_BITS)))
    t_bits = jnp.maximum(vk_bits, jnp.int32(THRESH_BITS))
    keep = pb_ref[...] < t_bits
    s = jnp.sum(jnp.where(keep, f_ref[0], 0.0))
    c = jnp.sum(keep.astype(jnp.float32))
    lane = lax.broadcasted_iota(jnp.int32, (1, 128), 1)
    out_ref[0] = jnp.where(lane == 0, s, jnp.where(lane == 1, c, 0.0))


def _fallback(p, f):
    in_spec = pl.BlockSpec((1, R, W), lambda j: (j, 0, 0))
    return pl.pallas_call(
        _fb_body,
        grid=(5,),
        in_specs=[in_spec, in_spec],
        out_specs=pl.BlockSpec((1, 1, 128), lambda j: (j, 0, 0)),
        out_shape=jax.ShapeDtypeStruct((5, 1, 128), jnp.float32),
        scratch_shapes=[pltpu.VMEM((R, W), jnp.int32)],
    )(p, f)


def kernel(x1, x2, x3, x4, x5, target):
    t = target.astype(jnp.int32).reshape(R, W)
    p, f = _dense_stage((x1, x2, x3, x4, x5), t)
    pr = _make_mine_sc()(p, f)
    res = _finish(pr)

    def _common():
        return res[0]

    def _rare():
        fb = _fallback(p, f)[:, 0, :2]
        s7 = res[2:7]
        c7 = res[7:12]
        loss = jnp.where(c7 > K, s7 / c7, fb[:, 0] / fb[:, 1])
        return jnp.sum(loss)

    return lax.cond(res[1] > 0.0, _rare, _common)


# speedup vs baseline: 1.1144x; 1.0109x over previous
import functools

import jax
import jax.numpy as jnp
import numpy as np
from jax import lax
from jax.experimental import pallas as pl
from jax.experimental.pallas import tpu as pltpu
from jax.experimental.pallas import tpu_sc as plsc

B, C, H, W = 4, 19, 512, 512
R = B * H
N = B * H * W
K = 100000
THRESH = np.float32(0.7)
THRESH_BITS = int(np.float32(0.7).view(np.int32))
ONE_BITS = int(np.float32(1.0).view(np.int32))


BH = 128
NHB = H // BH


def _dense_body(x1, x2, x3, x4, x5, t_ref, p_ref, f_ref):
    t = t_ref[...]
    cls = lax.broadcasted_iota(jnp.int32, (C, BH, W), 0)
    onehot = cls == t[None]
    ps = []
    fs = []
    for x in (x1, x2, x3, x4, x5):
        z = x[0]
        m = jnp.max(z, axis=0)
        s = jnp.sum(jnp.exp(z - m[None]), axis=0)
        lse = m + jnp.log(s)
        zt = jnp.sum(jnp.where(onehot, z, 0.0), axis=0)
        p = jnp.exp(zt - lse)
        ps.append(p)
        fs.append((1.0 - p) * (1.0 - p) * (lse - zt))
    p_ref[...] = jnp.stack(ps)
    f_ref[...] = jnp.stack(fs)


def _dense_stage(xs, t):
    x_spec = pl.BlockSpec((1, C, BH, W), lambda b, h: (b, 0, h, 0))
    t_spec = pl.BlockSpec((BH, W), lambda b, h: (b * NHB + h, 0))
    out_spec = pl.BlockSpec((5, BH, W), lambda b, h: (0, b * NHB + h, 0))
    out_shape = jax.ShapeDtypeStruct((5, R, W), jnp.float32)
    return pl.pallas_call(
        _dense_body,
        grid=(B, NHB),
        in_specs=[x_spec] * 5 + [t_spec],
        out_specs=[out_spec, out_spec],
        out_shape=[out_shape, out_shape],
    )(*xs, t)


NC, NS, L = 2, 16, 16
NW = NC * NS
WROWS = R // NW
CR = 16
NCH = WROWS // CR
NCHUNK = 5 * NCH


def _mine_sc_body(p_hbm, f_hbm, out_hbm, pb0, fb0, pb1, fb1, acc_v, sem0, sem1):
    wid = lax.axis_index("s") * NC + lax.axis_index("c")
    r0 = wid * WROWS
    bufs = ((pb0, fb0, sem0), (pb1, fb1, sem1))

    def chunk_src(idx):
        j = idx // NCH
        rr = r0 + (idx % NCH) * CR
        return p_hbm.at[j, pl.ds(rr, CR), :], f_hbm.at[j, pl.ds(rr, CR), :]

    def start(idx, par):
        psrc, fsrc = chunk_src(idx)
        pltpu.make_async_copy(psrc, bufs[par][0], bufs[par][2]).start()
        pltpu.make_async_copy(fsrc, bufs[par][1], bufs[par][2]).start()

    def wait(par):
        pltpu.make_async_copy(p_hbm.at[0, pl.ds(0, CR), :], bufs[par][0], bufs[par][2]).wait()
        pltpu.make_async_copy(f_hbm.at[0, pl.ds(0, CR), :], bufs[par][1], bufs[par][2]).wait()

    def compute(par, carry):
        pb, fb, _ = bufs[par]

        def vec_body(q, carry2):
            s7, c7 = carry2
            r = q >> 2
            cb = (q & 3) * 128
            for u in range(8):
                sl = pl.ds(cb + u * L, L)
                p = pb[r, sl]
                f = fb[r, sl]
                keep = p < THRESH
                s7 = s7 + jnp.where(keep, f, 0.0)
                c7 = c7 + jnp.where(keep, 1.0, 0.0)
            return s7, c7

        return lax.fori_loop(0, CR * 4, vec_body, carry)

    zero = jnp.zeros((L,), jnp.float32)
    start(0, 0)
    carry = (zero, zero)
    carrys = []
    for idx in range(NCHUNK):
        par = idx & 1
        start((idx + 1) % NCHUNK, 1 - par)
        wait(par)
        carry = compute(par, carry)
        if idx % NCH == NCH - 1:
            carrys.append(carry)
            carry = (zero, zero)
    wait(1 - (NCHUNK - 1) % 2)

    for j in range(5):
        acc_v[pl.ds(j * 2 * L, L)] = carrys[j][0]
        acc_v[pl.ds(j * 2 * L + L, L)] = carrys[j][1]
    pltpu.sync_copy(acc_v, out_hbm.at[pl.ds(wid * (5 * 2 * L), 5 * 2 * L)])


@functools.cache
def _make_mine_sc():
    mesh = plsc.VectorSubcoreMesh(
        core_axis_name="c", subcore_axis_name="s", num_cores=NC, num_subcores=NS
    )
    return functools.partial(
        pl.kernel,
        out_type=jax.ShapeDtypeStruct((NW * 5 * 2 * L,), jnp.float32),
        mesh=mesh,
        scratch_types=[
            pltpu.VMEM((CR, W), jnp.float32),
            pltpu.VMEM((CR, W), jnp.float32),
            pltpu.VMEM((CR, W), jnp.float32),
            pltpu.VMEM((CR, W), jnp.float32),
            pltpu.VMEM((5 * 2 * L,), jnp.float32),
            pltpu.SemaphoreType.DMA,
            pltpu.SemaphoreType.DMA,
        ],
    )(_mine_sc_body)


def _finish_body(pr_ref, out_ref):
    acc = pr_ref[...]
    loss = jnp.float32(0.0)
    need = jnp.float32(0.0)
    for j in range(5):
        s7 = jnp.sum(acc[:, j * 2 * L : j * 2 * L + L])
        c7 = jnp.sum(acc[:, j * 2 * L + L : (j + 1) * 2 * L])
        out_ref[2 + j] = s7
        out_ref[7 + j] = c7
        loss = loss + s7 / c7
        need = jnp.maximum(need, jnp.where(c7 <= K, 1.0, 0.0))
    out_ref[0] = loss
    out_ref[1] = need


def _finish(pr):
    return pl.pallas_call(
        _finish_body,
        in_specs=[pl.BlockSpec((NW, 5 * 2 * L), lambda: (0, 0))],
        out_specs=pl.BlockSpec(memory_space=pltpu.SMEM),
        out_shape=jax.ShapeDtypeStruct((16,), jnp.float32),
    )(pr.reshape(NW, 5 * 2 * L))


def _fb_body(p_ref, f_ref, out_ref, pb_ref):
    pb_ref[...] = lax.bitcast_convert_type(p_ref[0], jnp.int32)

    def bs_body(_, state):
        lo, hi = state
        mid = (lo + hi) // 2
        cnt = jnp.sum((pb_ref[...] <= mid).astype(jnp.int32))
        good = cnt >= K + 1
        return jnp.where(good, lo, mid + 1), jnp.where(good, mid, hi)

    _, vk_bits = lax.fori_loop(0, 31, bs_body, (jnp.int32(0), jnp.int32(ONE_BITS)))
    t_bits = jnp.maximum(vk_bits, jnp.int32(THRESH_BITS))
    keep = pb_ref[...] < t_bits
    s = jnp.sum(jnp.where(keep, f_ref[0], 0.0))
    c = jnp.sum(keep.astype(jnp.float32))
    lane = lax.broadcasted_iota(jnp.int32, (1, 128), 1)
    out_ref[0] = jnp.where(lane == 0, s, jnp.where(lane == 1, c, 0.0))


def _fallback(p, f):
    in_spec = pl.BlockSpec((1, R, W), lambda j: (j, 0, 0))
    return pl.pallas_call(
        _fb_body,
        grid=(5,),
        in_specs=[in_spec, in_spec],
        out_specs=pl.BlockSpec((1, 1, 128), lambda j: (j, 0, 0)),
        out_shape=jax.ShapeDtypeStruct((5, 1, 128), jnp.float32),
        scratch_shapes=[pltpu.VMEM((R, W), jnp.int32)],
    )(p, f)


def kernel(x1, x2, x3, x4, x5, target):
    t = target.astype(jnp.int32).reshape(R, W)
    p, f = _dense_stage((x1, x2, x3, x4, x5), t)
    pr = _make_mine_sc()(p, f)
    s7c7 = pr.reshape(NW, 5, 2, L).sum(axis=(0, 3))
    s7, c7 = s7c7[:, 0], s7c7[:, 1]

    def _common():
        return jnp.sum(s7 / c7)

    def _rare():
        fb = _fallback(p, f)[:, 0, :2]
        loss = jnp.where(c7 > K, s7 / c7, fb[:, 0] / fb[:, 1])
        return jnp.sum(loss)

    return lax.cond(jnp.any(c7 <= K), _rare, _common)


# speedup vs baseline: 1.1328x; 1.0165x over previous
import functools

import jax
import jax.numpy as jnp
import numpy as np
from jax import lax
from jax.experimental import pallas as pl
from jax.experimental.pallas import tpu as pltpu
from jax.experimental.pallas import tpu_sc as plsc

B, C, H, W = 4, 19, 512, 512
R = B * H
N = B * H * W
K = 100000
THRESH = np.float32(0.7)
THRESH_BITS = int(np.float32(0.7).view(np.int32))
ONE_BITS = int(np.float32(1.0).view(np.int32))


BH = 128
NHB = H // BH


def _dense_body(x1, x2, x3, x4, x5, t_ref, p_ref, f_ref):
    t = t_ref[...]
    cls = lax.broadcasted_iota(jnp.int32, (C, BH, W), 0)
    onehot = cls == t[None]
    ps = []
    fs = []
    for x in (x1, x2, x3, x4, x5):
        z = x[0]
        m = jnp.max(z, axis=0)
        s = jnp.sum(jnp.exp(z - m[None]), axis=0)
        lse = m + jnp.log(s)
        zt = jnp.sum(jnp.where(onehot, z, 0.0), axis=0)
        p = jnp.exp(zt - lse)
        ps.append(p)
        fs.append((1.0 - p) * (1.0 - p) * (lse - zt))
    p_ref[...] = jnp.stack(ps)
    f_ref[...] = jnp.stack(fs)


def _dense_stage(xs, t):
    x_spec = pl.BlockSpec((1, C, BH, W), lambda b, h: (b, 0, h, 0))
    t_spec = pl.BlockSpec((BH, W), lambda b, h: (b * NHB + h, 0))
    out_spec = pl.BlockSpec((5, BH, W), lambda b, h: (0, b * NHB + h, 0))
    out_shape = jax.ShapeDtypeStruct((5, R, W), jnp.float32)
    return pl.pallas_call(
        _dense_body,
        grid=(B, NHB),
        in_specs=[x_spec] * 5 + [t_spec],
        out_specs=[out_spec, out_spec],
        out_shape=[out_shape, out_shape],
    )(*xs, t)


NC, NS, L = 2, 16, 16
NW = NC * NS
WROWS = R // NW
CR = 32
NCH = WROWS // CR
NCHUNK = 5 * NCH


def _mine_sc_body(p_hbm, f_hbm, out_hbm, pb0, fb0, pb1, fb1, acc_v, sem0, sem1):
    wid = lax.axis_index("s") * NC + lax.axis_index("c")
    r0 = wid * WROWS
    bufs = ((pb0, fb0, sem0), (pb1, fb1, sem1))

    def chunk_src(idx):
        j = idx // NCH
        rr = r0 + (idx % NCH) * CR
        return p_hbm.at[j, pl.ds(rr, CR), :], f_hbm.at[j, pl.ds(rr, CR), :]

    def start(idx, par):
        psrc, fsrc = chunk_src(idx)
        pltpu.make_async_copy(psrc, bufs[par][0], bufs[par][2]).start()
        pltpu.make_async_copy(fsrc, bufs[par][1], bufs[par][2]).start()

    def wait(par):
        pltpu.make_async_copy(p_hbm.at[0, pl.ds(0, CR), :], bufs[par][0], bufs[par][2]).wait()
        pltpu.make_async_copy(f_hbm.at[0, pl.ds(0, CR), :], bufs[par][1], bufs[par][2]).wait()

    def compute(par, carry):
        pb, fb, _ = bufs[par]

        def vec_body(q, carry2):
            s7, c7 = carry2
            r = q >> 2
            cb = (q & 3) * 128
            for u in range(8):
                sl = pl.ds(cb + u * L, L)
                p = pb[r, sl]
                f = fb[r, sl]
                keep = p < THRESH
                s7 = s7 + jnp.where(keep, f, 0.0)
                c7 = c7 + jnp.where(keep, 1.0, 0.0)
            return s7, c7

        return lax.fori_loop(0, CR * 4, vec_body, carry)

    zero = jnp.zeros((L,), jnp.float32)
    start(0, 0)
    carry = (zero, zero)
    carrys = []
    for idx in range(NCHUNK):
        par = idx & 1
        start((idx + 1) % NCHUNK, 1 - par)
        wait(par)
        carry = compute(par, carry)
        if idx % NCH == NCH - 1:
            carrys.append(carry)
            carry = (zero, zero)
    wait(1 - (NCHUNK - 1) % 2)

    for j in range(5):
        acc_v[pl.ds(j * 2 * L, L)] = carrys[j][0]
        acc_v[pl.ds(j * 2 * L + L, L)] = carrys[j][1]
    pltpu.sync_copy(acc_v, out_hbm.at[pl.ds(wid * (5 * 2 * L), 5 * 2 * L)])


@functools.cache
def _make_mine_sc():
    mesh = plsc.VectorSubcoreMesh(
        core_axis_name="c", subcore_axis_name="s", num_cores=NC, num_subcores=NS
    )
    return functools.partial(
        pl.kernel,
        out_type=jax.ShapeDtypeStruct((NW * 5 * 2 * L,), jnp.float32),
        mesh=mesh,
        scratch_types=[
            pltpu.VMEM((CR, W), jnp.float32),
            pltpu.VMEM((CR, W), jnp.float32),
            pltpu.VMEM((CR, W), jnp.float32),
            pltpu.VMEM((CR, W), jnp.float32),
            pltpu.VMEM((5 * 2 * L,), jnp.float32),
            pltpu.SemaphoreType.DMA,
            pltpu.SemaphoreType.DMA,
        ],
    )(_mine_sc_body)


def _fb_body(p_ref, f_ref, out_ref, pb_ref):
    pb_ref[...] = lax.bitcast_convert_type(p_ref[0], jnp.int32)

    def bs_body(_, state):
        lo, hi = state
        mid = (lo + hi) // 2
        cnt = jnp.sum((pb_ref[...] <= mid).astype(jnp.int32))
        good = cnt >= K + 1
        return jnp.where(good, lo, mid + 1), jnp.where(good, mid, hi)

    _, vk_bits = lax.fori_loop(0, 31, bs_body, (jnp.int32(0), jnp.int32(ONE_BITS)))
    t_bits = jnp.maximum(vk_bits, jnp.int32(THRESH_BITS))
    keep = pb_ref[...] < t_bits
    s = jnp.sum(jnp.where(keep, f_ref[0], 0.0))
    c = jnp.sum(keep.astype(jnp.float32))
    lane = lax.broadcasted_iota(jnp.int32, (1, 128), 1)
    out_ref[0] = jnp.where(lane == 0, s, jnp.where(lane == 1, c, 0.0))


def _fallback(p, f):
    in_spec = pl.BlockSpec((1, R, W), lambda j: (j, 0, 0))
    return pl.pallas_call(
        _fb_body,
        grid=(5,),
        in_specs=[in_spec, in_spec],
        out_specs=pl.BlockSpec((1, 1, 128), lambda j: (j, 0, 0)),
        out_shape=jax.ShapeDtypeStruct((5, 1, 128), jnp.float32),
        scratch_shapes=[pltpu.VMEM((R, W), jnp.int32)],
    )(p, f)


def kernel(x1, x2, x3, x4, x5, target):
    t = target.astype(jnp.int32).reshape(R, W)
    p, f = _dense_stage((x1, x2, x3, x4, x5), t)
    pr = _make_mine_sc()(p, f)
    s7c7 = pr.reshape(NW, 5, 2, L).sum(axis=(0, 3))
    s7, c7 = s7c7[:, 0], s7c7[:, 1]

    def _common():
        return jnp.sum(s7 / c7)

    def _rare():
        fb = _fallback(p, f)[:, 0, :2]
        loss = jnp.where(c7 > K, s7 / c7, fb[:, 0] / fb[:, 1])
        return jnp.sum(loss)

    return lax.cond(jnp.any(c7 <= K), _rare, _common)
